# baseline (speedup 1.0000x reference)
import functools

import jax
import jax.numpy as jnp
from jax import lax
from jax.experimental import pallas as pl
from jax.experimental.pallas import tpu as pltpu
from jax.experimental.pallas import tpu_sc as plsc

D = 2048
E = 8
K = 2
T = 8192
TK = T * K
BM = 512
P = TK + E * BM
NB = P // BM

NC = 2
NS = 16
NW = NC * NS
TOK_W = T // NW

def _pack_halves(x):
    u = lax.bitcast_convert_type(x, jnp.uint32)
    r = (u + jnp.uint32(0x7FFF) + ((u >> 16) & jnp.uint32(1))) >> 16
    n = x.shape[1] // 2
    return lax.bitcast_convert_type(r[:, :n] | (r[:, n:] << 16), jnp.float32)


def _unpack_halves(pk):
    u = lax.bitcast_convert_type(pk, jnp.uint32)
    lo = lax.bitcast_convert_type(u << 16, jnp.float32)
    hi = lax.bitcast_convert_type(u & jnp.uint32(0xFFFF0000), jnp.float32)
    return jnp.concatenate([lo, hi], axis=1)


_RT = 512


def _router_body(x_ref, gw_ref, gb_ref, logits_ref, sel_ref, w_ref, rank_ref,
                 counts_ref, xbf_ref, cacc_ref):
    i = pl.program_id(0)

    @pl.when(i == 0)
    def _():
        cacc_ref[...] = jnp.zeros_like(cacc_ref)

    x = x_ref[...]
    xbf_ref[...] = _pack_halves(x)
    logits = jnp.dot(x, gw_ref[...], preferred_element_type=jnp.float32)
    logits = logits + gb_ref[...]
    logits_ref[...] = logits
    m = jnp.max(logits, axis=-1, keepdims=True)
    ex = jnp.exp(logits - m)
    probs = ex / jnp.sum(ex, axis=-1, keepdims=True)
    iota = lax.broadcasted_iota(jnp.int32, probs.shape, 1)
    m1 = jnp.max(probs, axis=-1, keepdims=True)
    a1 = jnp.min(jnp.where(probs == m1, iota, E), axis=-1)
    masked = jnp.where(iota == a1[:, None], -1.0, probs)
    m2 = jnp.max(masked, axis=-1, keepdims=True)
    a2 = jnp.min(jnp.where(masked == m2, iota, E), axis=-1)
    s = m1 + m2
    sel_ref[...] = jnp.stack([a1, a2], axis=-1).astype(jnp.int32)
    w_ref[...] = jnp.concatenate([m1 / s, m2 / s], axis=-1)
    oh0 = (iota == a1[:, None]).astype(jnp.float32)
    oh1 = (iota == a2[:, None]).astype(jnp.float32)
    mm = oh0 + oh1
    r_i = lax.broadcasted_iota(jnp.int32, (_RT, _RT), 0)
    c_i = lax.broadcasted_iota(jnp.int32, (_RT, _RT), 1)
    tri = (c_i < r_i).astype(jnp.float32)
    pre = jnp.dot(tri, mm, preferred_element_type=jnp.float32)
    tot = pre + cacc_ref[...]
    r0 = jnp.sum(tot * oh0, axis=1)
    r1 = jnp.sum(tot * oh1, axis=1)
    rank_ref[...] = jnp.stack([r0, r1], axis=-1).astype(jnp.int32)
    newc = cacc_ref[...] + jnp.sum(mm, axis=0, keepdims=True)
    cacc_ref[...] = newc
    counts_ref[...] = newc.astype(jnp.int32)


def _router(xf, gate_W, gate_b2):
    return pl.pallas_call(
        _router_body,
        grid=(T // _RT,),
        in_specs=[
            pl.BlockSpec((_RT, D), lambda i: (i, 0)),
            pl.BlockSpec((D, E), lambda i: (0, 0)),
            pl.BlockSpec((1, E), lambda i: (0, 0)),
        ],
        out_specs=[
            pl.BlockSpec((_RT, E), lambda i: (i, 0)),
            pl.BlockSpec((_RT, K), lambda i: (i, 0)),
            pl.BlockSpec((_RT, K), lambda i: (i, 0)),
            pl.BlockSpec((_RT, K), lambda i: (i, 0)),
            pl.BlockSpec((1, E), lambda i: (0, 0)),
            pl.BlockSpec((_RT, D // 2), lambda i: (i, 0)),
        ],
        out_shape=[
            jax.ShapeDtypeStruct((T, E), jnp.float32),
            jax.ShapeDtypeStruct((T, K), jnp.int32),
            jax.ShapeDtypeStruct((T, K), jnp.float32),
            jax.ShapeDtypeStruct((T, K), jnp.int32),
            jax.ShapeDtypeStruct((1, E), jnp.int32),
            jax.ShapeDtypeStruct((T, D // 2), jnp.float32),
        ],
        scratch_shapes=[pltpu.VMEM((1, E), jnp.float32)],
    )(xf, gate_W, gate_b2)


def _dispatch_metadata(sel, rank, counts):
    pc = ((counts + BM - 1) // BM) * BM
    poff = jnp.concatenate([jnp.zeros(1, jnp.int32), jnp.cumsum(pc)[:-1].astype(jnp.int32)])
    slot2 = poff[sel] + rank
    block_starts = jnp.arange(NB, dtype=jnp.int32) * BM
    block_expert = (jnp.searchsorted(poff, block_starts, side="right") - 1).astype(jnp.int32)
    return slot2, block_expert


_D_CH = 32
_D_NCH = TOK_W // _D_CH


def _sc_index_body(p0_hbm, p1_hbm, w0_hbm, w1_hbm, ws_hbm, rid_hbm,
                   i0a, i1a, w0a, w1a, tv, sem):
    wid = lax.axis_index("s") * NC + lax.axis_index("c")
    cb = wid * _D_NCH
    base = wid * TOK_W
    pltpu.sync_copy(p0_hbm.at[pl.ds(cb, _D_NCH)], i0a)
    pltpu.sync_copy(p1_hbm.at[pl.ds(cb, _D_NCH)], i1a)
    pltpu.sync_copy(w0_hbm.at[pl.ds(cb, _D_NCH)], w0a)
    pltpu.sync_copy(w1_hbm.at[pl.ds(cb, _D_NCH)], w1a)
    iot = lax.iota(jnp.int32, 16)
    for g in range(_D_NCH):
        for j in range(_D_CH // 16):
            tv[g, 0, pl.ds(j * 16, 16)] = base + g * _D_CH + j * 16 + iot
    cps = []
    for g in range(_D_NCH):
        cps.append(pltpu.async_copy(w0a.at[g, 0], ws_hbm.at[i0a.at[g, 0]], sem))
        cps.append(pltpu.async_copy(w1a.at[g, 0], ws_hbm.at[i1a.at[g, 0]], sem))
        cps.append(pltpu.async_copy(tv.at[g, 0], rid_hbm.at[i0a.at[g, 0]], sem))
        cps.append(pltpu.async_copy(tv.at[g, 0], rid_hbm.at[i1a.at[g, 0]], sem))
    for c in cps:
        c.wait()


def _sc_index(p0r, p1r, w0r, w1r):
    mesh = plsc.VectorSubcoreMesh(core_axis_name="c", subcore_axis_name="s")
    return pl.kernel(
        _sc_index_body,
        out_type=[
            jax.ShapeDtypeStruct((P,), jnp.float32),
            jax.ShapeDtypeStruct((P,), jnp.int32),
        ],
        mesh=mesh,
        scratch_types=[
            pltpu.VMEM((_D_NCH, 1, _D_CH), jnp.int32),
            pltpu.VMEM((_D_NCH, 1, _D_CH), jnp.int32),
            pltpu.VMEM((_D_NCH, 1, _D_CH), jnp.float32),
            pltpu.VMEM((_D_NCH, 1, _D_CH), jnp.float32),
            pltpu.VMEM((_D_NCH, 1, _D_CH), jnp.int32),
            pltpu.SemaphoreType.DMA,
        ],
    )(p0r, p1r, w0r, w1r)


_G_CH = 32
_G_ROWS = P // NW
_G_NCH = _G_ROWS // _G_CH


def _sc_gather_body(x_hbm, rid_hbm, xs_hbm, ia, xb0, xb1, gs0, gs1, ws0, ws1):
    wid = lax.axis_index("s") * NC + lax.axis_index("c")
    base = wid * _G_ROWS
    cb = wid * _G_NCH
    pltpu.sync_copy(rid_hbm.at[pl.ds(cb, _G_NCH)], ia)
    for g in range(_G_NCH):
        for j in range(_G_CH // 16):
            sl = pl.ds(j * 16, 16)
            ia[g, 0, sl] = jnp.minimum(jnp.maximum(ia[g, 0, sl], 0), T - 1)
    xbs = (xb0, xb1)
    gss = (gs0, gs1)
    wss = (ws0, ws1)
    pltpu.async_copy(x_hbm.at[ia.at[0, 0]], xb0, gs0)
    pltpu.async_copy(x_hbm.at[ia.at[1, 0]], xb1, gs1)

    @pl.loop(0, _G_NCH, step=2)
    def _outer(g0):
        for b in range(2):
            g = g0 + b
            xb, gs, ws = xbs[b], gss[b], wss[b]
            pltpu.make_async_copy(x_hbm.at[ia.at[0, 0]], xb, gs).wait()
            pltpu.async_copy(xb, xs_hbm.at[pl.ds(base + g * _G_CH, _G_CH)], ws)

            @pl.when(g + 2 < _G_NCH)
            def _():
                pltpu.make_async_copy(
                    xb, xs_hbm.at[pl.ds(base, _G_CH)], ws).wait()
                pltpu.async_copy(x_hbm.at[ia.at[g + 2, 0]], xb, gs)

    for b in range(2):
        pltpu.make_async_copy(xbs[b], xs_hbm.at[pl.ds(base, _G_CH)],
                              wss[b]).wait()


def _sc_gather(xpk, rid_r):
    mesh = plsc.VectorSubcoreMesh(core_axis_name="c", subcore_axis_name="s")
    return pl.kernel(
        _sc_gather_body,
        out_type=jax.ShapeDtypeStruct((P, D // 2), jnp.float32),
        mesh=mesh,
        scratch_types=[
            pltpu.VMEM((_G_NCH, 1, _G_CH), jnp.int32),
            pltpu.VMEM((_G_CH, D // 2), jnp.float32),
            pltpu.VMEM((_G_CH, D // 2), jnp.float32),
            pltpu.SemaphoreType.DMA,
            pltpu.SemaphoreType.DMA,
            pltpu.SemaphoreType.DMA,
            pltpu.SemaphoreType.DMA,
        ],
    )(xpk, rid_r)


def _gmm_body(be_ref, x_ref, w_ref, W_ref, b_ref, o_ref):
    xb = _unpack_halves(x_ref[...])
    acc = jnp.dot(xb, W_ref[0], preferred_element_type=jnp.float32)
    y = (acc + b_ref[0, 0][None, :]) * w_ref[0, 0][:, None]
    o_ref[...] = _pack_halves(y)


def _gmm(block_expert, x_sorted, w3, expert_W, expert_b3):
    grid_spec = pltpu.PrefetchScalarGridSpec(
        num_scalar_prefetch=1,
        grid=(NB,),
        in_specs=[
            pl.BlockSpec((BM, D // 2), lambda i, be: (i, 0)),
            pl.BlockSpec((1, 1, BM), lambda i, be: (i, 0, 0)),
            pl.BlockSpec((1, D, D), lambda i, be: (be[i], 0, 0)),
            pl.BlockSpec((1, 1, D), lambda i, be: (be[i], 0, 0)),
        ],
        out_specs=pl.BlockSpec((BM, D // 2), lambda i, be: (i, 0)),
    )
    return pl.pallas_call(
        _gmm_body,
        grid_spec=grid_spec,
        out_shape=jax.ShapeDtypeStruct((P, D // 2), jnp.float32),
    )(block_expert, x_sorted, w3, expert_W, expert_b3)


_C_CH = 8
_C_NCH = TOK_W // _C_CH
_C_R = 2 * _C_CH


def _sc_combine_body(y_hbm, sl_hbm, out_hbm, ia, yb0, yb1, ob0, ob1,
                     gsem0, gsem1, osem0, osem1):
    wid = lax.axis_index("s") * NC + lax.axis_index("c")
    base = wid * TOK_W
    cb = wid * _C_NCH
    pltpu.sync_copy(sl_hbm.at[pl.ds(cb, _C_NCH)], ia)
    ybufs = (yb0, yb1)
    obufs = (ob0, ob1)
    gsems = (gsem0, gsem1)
    osems = (osem0, osem1)
    pltpu.async_copy(y_hbm.at[ia.at[0, 0]], yb0, gsem0)
    pltpu.async_copy(y_hbm.at[ia.at[1, 0]], yb1, gsem1)

    @pl.loop(0, _C_NCH, step=2)
    def _outer(g0):
        for b in range(2):
            g = g0 + b
            yb, ob, gsem, osem = ybufs[b], obufs[b], gsems[b], osems[b]
            pltpu.make_async_copy(y_hbm.at[ia.at[0, 0]], yb, gsem).wait()
            @pl.when(g >= 2)
            def _():
                pltpu.make_async_copy(ob, out_hbm.at[pl.ds(base, _C_CH)],
                                      osem).wait()
            for r in range(_C_CH):
                @plsc.parallel_loop(0, D // 32, unroll=8)
                def _adds(j):
                    sl = pl.ds(j * 16, 16)
                    bc = lax.bitcast_convert_type
                    u0 = bc(yb[2 * r, sl], jnp.uint32)
                    u1 = bc(yb[2 * r + 1, sl], jnp.uint32)
                    lo = bc(u0 << 16, jnp.float32) + bc(u1 << 16, jnp.float32)
                    hm = jnp.uint32(0xFFFF0000)
                    hi = bc(u0 & hm, jnp.float32) + bc(u1 & hm, jnp.float32)
                    ob[r, sl] = lo
                    ob[r, pl.ds(D // 2 + j * 16, 16)] = hi
            pltpu.async_copy(ob, out_hbm.at[pl.ds(base + g * _C_CH, _C_CH)],
                             osem)

            @pl.when(g + 2 < _C_NCH)
            def _():
                pltpu.async_copy(y_hbm.at[ia.at[g + 2, 0]], yb, gsem)

    pltpu.make_async_copy(ob0, out_hbm.at[pl.ds(base, _C_CH)], osem0).wait()
    pltpu.make_async_copy(ob1, out_hbm.at[pl.ds(base, _C_CH)], osem1).wait()


def _sc_combine(y_sorted, slot3):
    mesh = plsc.VectorSubcoreMesh(core_axis_name="c", subcore_axis_name="s")
    return pl.kernel(
        _sc_combine_body,
        out_type=jax.ShapeDtypeStruct((T, D), jnp.float32),
        mesh=mesh,
        scratch_types=[
            pltpu.VMEM((_C_NCH, 1, _C_R), jnp.int32),
            pltpu.VMEM((_C_R, D // 2), jnp.float32),
            pltpu.VMEM((_C_R, D // 2), jnp.float32),
            pltpu.VMEM((_C_CH, D), jnp.float32),
            pltpu.VMEM((_C_CH, D), jnp.float32),
            pltpu.SemaphoreType.DMA,
            pltpu.SemaphoreType.DMA,
            pltpu.SemaphoreType.DMA,
            pltpu.SemaphoreType.DMA,
        ],
    )(y_sorted, slot3)


def kernel(x, gate_W, gate_b, expert_W, expert_b):
    b, s, d = x.shape
    xf = x.reshape(T, D)
    logits, sel, w, rank, counts, xpk = _router(xf, gate_W, gate_b.reshape(1, E))
    slot2, block_expert = _dispatch_metadata(sel, rank, counts[0])
    p0r = slot2[:, 0].reshape(T // _D_CH, 1, _D_CH)
    p1r = slot2[:, 1].reshape(T // _D_CH, 1, _D_CH)
    w0r = w[:, 0].reshape(T // _D_CH, 1, _D_CH)
    w1r = w[:, 1].reshape(T // _D_CH, 1, _D_CH)
    w_sorted, rid = _sc_index(p0r, p1r, w0r, w1r)
    xs_pk = _sc_gather(xpk, rid.reshape(P // _G_CH, 1, _G_CH))
    y_sorted = _gmm(block_expert, xs_pk, w_sorted.reshape(NB, 1, BM),
                    expert_W, expert_b.reshape(E, 1, D))
    slot3 = slot2.reshape(T // _C_CH, 1, _C_R)
    out = _sc_combine(y_sorted, slot3)
    return out.reshape(b, s, d), logits, sel


# speedup vs baseline: 1.5951x; 1.5951x over previous
import functools

import jax
import jax.numpy as jnp
from jax import lax
from jax.experimental import pallas as pl
from jax.experimental.pallas import tpu as pltpu
from jax.experimental.pallas import tpu_sc as plsc

D = 2048
E = 8
K = 2
T = 8192
TK = T * K
BM = 512
P = TK + E * BM
NB = P // BM

NC = 2
NS = 16
NW = NC * NS
TOK_W = T // NW

def _pack_halves(x):
    u = lax.bitcast_convert_type(x, jnp.uint32)
    r = (u + jnp.uint32(0x7FFF) + ((u >> 16) & jnp.uint32(1))) >> 16
    n = x.shape[1] // 2
    return lax.bitcast_convert_type(r[:, :n] | (r[:, n:] << 16), jnp.float32)


def _unpack_halves(pk):
    u = lax.bitcast_convert_type(pk, jnp.uint32)
    lo = lax.bitcast_convert_type(u << 16, jnp.float32)
    hi = lax.bitcast_convert_type(u & jnp.uint32(0xFFFF0000), jnp.float32)
    return jnp.concatenate([lo, hi], axis=1)


_RT = 512


def _router_body(x_ref, gw_ref, gb_ref, logits_ref, sel_ref, w_ref, rank_ref,
                 counts_ref, xbf_ref, cacc_ref):
    i = pl.program_id(0)

    @pl.when(i == 0)
    def _():
        cacc_ref[...] = jnp.zeros_like(cacc_ref)

    x = x_ref[...]
    xbf_ref[...] = _pack_halves(x)
    logits = jnp.dot(x, gw_ref[...], preferred_element_type=jnp.float32)
    logits = logits + gb_ref[...]
    logits_ref[...] = logits
    m = jnp.max(logits, axis=-1, keepdims=True)
    ex = jnp.exp(logits - m)
    probs = ex / jnp.sum(ex, axis=-1, keepdims=True)
    iota = lax.broadcasted_iota(jnp.int32, probs.shape, 1)
    m1 = jnp.max(probs, axis=-1, keepdims=True)
    a1 = jnp.min(jnp.where(probs == m1, iota, E), axis=-1)
    masked = jnp.where(iota == a1[:, None], -1.0, probs)
    m2 = jnp.max(masked, axis=-1, keepdims=True)
    a2 = jnp.min(jnp.where(masked == m2, iota, E), axis=-1)
    s = m1 + m2
    sel_ref[...] = jnp.stack([a1, a2], axis=-1).astype(jnp.int32)
    w_ref[...] = jnp.concatenate([m1 / s, m2 / s], axis=-1)
    oh0 = (iota == a1[:, None]).astype(jnp.float32)
    oh1 = (iota == a2[:, None]).astype(jnp.float32)
    mm = oh0 + oh1
    r_i = lax.broadcasted_iota(jnp.int32, (_RT, _RT), 0)
    c_i = lax.broadcasted_iota(jnp.int32, (_RT, _RT), 1)
    tri = (c_i < r_i).astype(jnp.float32)
    pre = jnp.dot(tri, mm, preferred_element_type=jnp.float32)
    tot = pre + cacc_ref[...]
    r0 = jnp.sum(tot * oh0, axis=1)
    r1 = jnp.sum(tot * oh1, axis=1)
    rank_ref[...] = jnp.stack([r0, r1], axis=-1).astype(jnp.int32)
    newc = cacc_ref[...] + jnp.sum(mm, axis=0, keepdims=True)
    cacc_ref[...] = newc
    counts_ref[...] = newc.astype(jnp.int32)


def _router(xf, gate_W, gate_b2):
    return pl.pallas_call(
        _router_body,
        grid=(T // _RT,),
        in_specs=[
            pl.BlockSpec((_RT, D), lambda i: (i, 0)),
            pl.BlockSpec((D, E), lambda i: (0, 0)),
            pl.BlockSpec((1, E), lambda i: (0, 0)),
        ],
        out_specs=[
            pl.BlockSpec((_RT, E), lambda i: (i, 0)),
            pl.BlockSpec((_RT, K), lambda i: (i, 0)),
            pl.BlockSpec((_RT, K), lambda i: (i, 0)),
            pl.BlockSpec((_RT, K), lambda i: (i, 0)),
            pl.BlockSpec((1, E), lambda i: (0, 0)),
            pl.BlockSpec((_RT, D // 2), lambda i: (i, 0)),
        ],
        out_shape=[
            jax.ShapeDtypeStruct((T, E), jnp.float32),
            jax.ShapeDtypeStruct((T, K), jnp.int32),
            jax.ShapeDtypeStruct((T, K), jnp.float32),
            jax.ShapeDtypeStruct((T, K), jnp.int32),
            jax.ShapeDtypeStruct((1, E), jnp.int32),
            jax.ShapeDtypeStruct((T, D // 2), jnp.float32),
        ],
        scratch_shapes=[pltpu.VMEM((1, E), jnp.float32)],
    )(xf, gate_W, gate_b2)


def _dispatch_metadata(sel, rank, counts):
    pc = ((counts + BM - 1) // BM) * BM
    poff = jnp.concatenate([jnp.zeros(1, jnp.int32), jnp.cumsum(pc)[:-1].astype(jnp.int32)])
    slot2 = poff[sel] + rank
    block_starts = jnp.arange(NB, dtype=jnp.int32) * BM
    block_expert = (jnp.searchsorted(poff, block_starts, side="right") - 1).astype(jnp.int32)
    return slot2, block_expert


_D_CH = 32
_D_NCH = TOK_W // _D_CH


def _sc_dispatch_body(x_hbm, p0_hbm, p1_hbm, w0_hbm, w1_hbm, xs_hbm, ws_hbm,
                      i0a, i1a, w0a, w1a, xb0, xb1, rs0, rs1, ss0, ss1):
    wid = lax.axis_index("s") * NC + lax.axis_index("c")
    cb = wid * _D_NCH
    base = wid * TOK_W
    pltpu.sync_copy(p0_hbm.at[pl.ds(cb, _D_NCH)], i0a)
    pltpu.sync_copy(p1_hbm.at[pl.ds(cb, _D_NCH)], i1a)
    pltpu.sync_copy(w0_hbm.at[pl.ds(cb, _D_NCH)], w0a)
    pltpu.sync_copy(w1_hbm.at[pl.ds(cb, _D_NCH)], w1a)
    xbs = (xb0, xb1)
    rss = (rs0, rs1)
    sss = (ss0, ss1)
    for i in range(2):
        pltpu.async_copy(x_hbm.at[pl.ds(base + i * _D_CH, _D_CH)],
                         xbs[i], rss[i])

    @pl.loop(0, _D_NCH, step=2)
    def _outer(g0):
        for b in range(2):
            g = g0 + b
            xb, rs, ss = xbs[b], rss[b], sss[b]
            pltpu.make_async_copy(x_hbm.at[pl.ds(base, _D_CH)], xb, rs).wait()
            pltpu.async_copy(xb, xs_hbm.at[i0a.at[g, 0]], ss)
            pltpu.async_copy(xb, xs_hbm.at[i1a.at[g, 0]], ss)
            pltpu.async_copy(w0a.at[g, 0], ws_hbm.at[i0a.at[g, 0]], ss)
            pltpu.async_copy(w1a.at[g, 0], ws_hbm.at[i1a.at[g, 0]], ss)

            @pl.when(g + 2 < _D_NCH)
            def _():
                pltpu.make_async_copy(xb, xs_hbm.at[i0a.at[g, 0]], ss).wait()
                pltpu.make_async_copy(xb, xs_hbm.at[i1a.at[g, 0]], ss).wait()
                pltpu.make_async_copy(w0a.at[g, 0], ws_hbm.at[i0a.at[g, 0]],
                                      ss).wait()
                pltpu.make_async_copy(w1a.at[g, 0], ws_hbm.at[i1a.at[g, 0]],
                                      ss).wait()
                pltpu.async_copy(
                    x_hbm.at[pl.ds(base + (g + 2) * _D_CH, _D_CH)], xb, rs)

    for b in range(2):
        g = _D_NCH - 2 + b
        pltpu.make_async_copy(xbs[b], xs_hbm.at[i0a.at[g, 0]], sss[b]).wait()
        pltpu.make_async_copy(xbs[b], xs_hbm.at[i1a.at[g, 0]], sss[b]).wait()
        pltpu.make_async_copy(w0a.at[g, 0], ws_hbm.at[i0a.at[g, 0]],
                              sss[b]).wait()
        pltpu.make_async_copy(w1a.at[g, 0], ws_hbm.at[i1a.at[g, 0]],
                              sss[b]).wait()


def _sc_dispatch(xf, p0r, p1r, w0r, w1r):
    mesh = plsc.VectorSubcoreMesh(core_axis_name="c", subcore_axis_name="s")
    return pl.kernel(
        _sc_dispatch_body,
        out_type=[
            jax.ShapeDtypeStruct((P, D // 2), jnp.float32),
            jax.ShapeDtypeStruct((P,), jnp.float32),
        ],
        mesh=mesh,
        scratch_types=[
            pltpu.VMEM((_D_NCH, 1, _D_CH), jnp.int32),
            pltpu.VMEM((_D_NCH, 1, _D_CH), jnp.int32),
            pltpu.VMEM((_D_NCH, 1, _D_CH), jnp.float32),
            pltpu.VMEM((_D_NCH, 1, _D_CH), jnp.float32),
            pltpu.VMEM((_D_CH, D // 2), jnp.float32),
            pltpu.VMEM((_D_CH, D // 2), jnp.float32),
            pltpu.SemaphoreType.DMA,
            pltpu.SemaphoreType.DMA,
            pltpu.SemaphoreType.DMA,
            pltpu.SemaphoreType.DMA,
        ],
    )(xf, p0r, p1r, w0r, w1r)


def _gmm_body(be_ref, x_ref, w_ref, W_ref, b_ref, o_ref):
    xb = _unpack_halves(x_ref[...])
    acc = jnp.dot(xb, W_ref[0], preferred_element_type=jnp.float32)
    y = (acc + b_ref[0, 0][None, :]) * w_ref[0, 0][:, None]
    o_ref[...] = _pack_halves(y)


def _gmm(block_expert, x_sorted, w3, expert_W, expert_b3):
    grid_spec = pltpu.PrefetchScalarGridSpec(
        num_scalar_prefetch=1,
        grid=(NB,),
        in_specs=[
            pl.BlockSpec((BM, D // 2), lambda i, be: (i, 0)),
            pl.BlockSpec((1, 1, BM), lambda i, be: (i, 0, 0)),
            pl.BlockSpec((1, D, D), lambda i, be: (be[i], 0, 0)),
            pl.BlockSpec((1, 1, D), lambda i, be: (be[i], 0, 0)),
        ],
        out_specs=pl.BlockSpec((BM, D // 2), lambda i, be: (i, 0)),
    )
    return pl.pallas_call(
        _gmm_body,
        grid_spec=grid_spec,
        out_shape=jax.ShapeDtypeStruct((P, D // 2), jnp.float32),
    )(block_expert, x_sorted, w3, expert_W, expert_b3)


_C_CH = 8
_C_NCH = TOK_W // _C_CH
_C_R = 2 * _C_CH


def _sc_combine_body(y_hbm, sl_hbm, out_hbm, ia, yb0, yb1, ob0, ob1,
                     gsem0, gsem1, osem0, osem1):
    wid = lax.axis_index("s") * NC + lax.axis_index("c")
    base = wid * TOK_W
    cb = wid * _C_NCH
    pltpu.sync_copy(sl_hbm.at[pl.ds(cb, _C_NCH)], ia)
    ybufs = (yb0, yb1)
    obufs = (ob0, ob1)
    gsems = (gsem0, gsem1)
    osems = (osem0, osem1)
    pltpu.async_copy(y_hbm.at[ia.at[0, 0]], yb0, gsem0)
    pltpu.async_copy(y_hbm.at[ia.at[1, 0]], yb1, gsem1)

    @pl.loop(0, _C_NCH, step=2)
    def _outer(g0):
        for b in range(2):
            g = g0 + b
            yb, ob, gsem, osem = ybufs[b], obufs[b], gsems[b], osems[b]
            pltpu.make_async_copy(y_hbm.at[ia.at[0, 0]], yb, gsem).wait()
            @pl.when(g >= 2)
            def _():
                pltpu.make_async_copy(ob, out_hbm.at[pl.ds(base, _C_CH)],
                                      osem).wait()
            for r in range(_C_CH):
                @plsc.parallel_loop(0, D // 32, unroll=8)
                def _adds(j):
                    sl = pl.ds(j * 16, 16)
                    bc = lax.bitcast_convert_type
                    u0 = bc(yb[2 * r, sl], jnp.uint32)
                    u1 = bc(yb[2 * r + 1, sl], jnp.uint32)
                    lo = bc(u0 << 16, jnp.float32) + bc(u1 << 16, jnp.float32)
                    hm = jnp.uint32(0xFFFF0000)
                    hi = bc(u0 & hm, jnp.float32) + bc(u1 & hm, jnp.float32)
                    ob[r, sl] = lo
                    ob[r, pl.ds(D // 2 + j * 16, 16)] = hi
            pltpu.async_copy(ob, out_hbm.at[pl.ds(base + g * _C_CH, _C_CH)],
                             osem)

            @pl.when(g + 2 < _C_NCH)
            def _():
                pltpu.async_copy(y_hbm.at[ia.at[g + 2, 0]], yb, gsem)

    pltpu.make_async_copy(ob0, out_hbm.at[pl.ds(base, _C_CH)], osem0).wait()
    pltpu.make_async_copy(ob1, out_hbm.at[pl.ds(base, _C_CH)], osem1).wait()


def _sc_combine(y_sorted, slot3):
    mesh = plsc.VectorSubcoreMesh(core_axis_name="c", subcore_axis_name="s")
    return pl.kernel(
        _sc_combine_body,
        out_type=jax.ShapeDtypeStruct((T, D), jnp.float32),
        mesh=mesh,
        scratch_types=[
            pltpu.VMEM((_C_NCH, 1, _C_R), jnp.int32),
            pltpu.VMEM((_C_R, D // 2), jnp.float32),
            pltpu.VMEM((_C_R, D // 2), jnp.float32),
            pltpu.VMEM((_C_CH, D), jnp.float32),
            pltpu.VMEM((_C_CH, D), jnp.float32),
            pltpu.SemaphoreType.DMA,
            pltpu.SemaphoreType.DMA,
            pltpu.SemaphoreType.DMA,
            pltpu.SemaphoreType.DMA,
        ],
    )(y_sorted, slot3)


def kernel(x, gate_W, gate_b, expert_W, expert_b):
    b, s, d = x.shape
    xf = x.reshape(T, D)
    logits, sel, w, rank, counts, xpk = _router(xf, gate_W, gate_b.reshape(1, E))
    slot2, block_expert = _dispatch_metadata(sel, rank, counts[0])
    p0r = slot2[:, 0].reshape(T // _D_CH, 1, _D_CH)
    p1r = slot2[:, 1].reshape(T // _D_CH, 1, _D_CH)
    w0r = w[:, 0].reshape(T // _D_CH, 1, _D_CH)
    w1r = w[:, 1].reshape(T // _D_CH, 1, _D_CH)
    xs_pk, w_sorted = _sc_dispatch(xpk, p0r, p1r, w0r, w1r)
    y_sorted = _gmm(block_expert, xs_pk, w_sorted.reshape(NB, 1, BM),
                    expert_W, expert_b.reshape(E, 1, D))
    slot3 = slot2.reshape(T // _C_CH, 1, _C_R)
    out = _sc_combine(y_sorted, slot3)
    return out.reshape(b, s, d), logits, sel
